# baseline (speedup 1.0000x reference)
import functools
import math

import jax
import jax.numpy as jnp
from jax import lax
from jax.experimental import pallas as pl
from jax.experimental.pallas import tpu as pltpu
from jax.experimental.pallas import tpu_sc as plsc

B = 2
S = 2048
D = 1024
DFF = 4096
E = 16
TOPK = 2
T = B * S
C = 512
EC = E * C

NC = 2
NS = 16
NW = NC * NS
RPW = EC // NW
GCH = 64
TPW = T // NW
CCH = 16

FB = 2048
NF = DFF // FB

_INV_SQRT2 = 1.0 / math.sqrt(2.0)


def _router_body(xf_ref, epsT_ref, wt_ref, bt_ref, meta_ref, mask_ref, cnt_ref):
    xf = xf_ref[...]
    wt = wt_ref[...]
    lgT = lax.dot_general(wt, xf, (((1,), (1,)), ((), ())),
                          preferred_element_type=jnp.float32)
    lgT = lgT + bt_ref[...]
    logits = lgT[0:E, :]
    nlog = lgT[E:2 * E, :]
    sp = jnp.maximum(nlog, 0.0) + jnp.log1p(jnp.exp(-jnp.abs(nlog)))
    noisy = logits + epsT_ref[...] * sp

    row = lax.broadcasted_iota(jnp.int32, (E, T), 0)
    m1 = jnp.max(noisy, axis=0, keepdims=True)
    i1 = jnp.min(jnp.where(noisy == m1, row, E), axis=0, keepdims=True)
    noisy2 = jnp.where(row == i1, -jnp.inf, noisy)
    m2 = jnp.max(noisy2, axis=0, keepdims=True)
    i2 = jnp.min(jnp.where(noisy2 == m2, row, E), axis=0, keepdims=True)

    e21 = jnp.exp(m2 - m1)
    den = 1.0 + e21
    g1 = 1.0 / den
    g2 = e21 / den

    mask_ref[...] = ((row == i1) | (row == i2)).astype(jnp.float32)

    r5 = lax.broadcasted_iota(jnp.int32, (512, 512), 0)
    c5 = lax.broadcasted_iota(jnp.int32, (512, 512), 1)
    tri = (r5 < c5).astype(jnp.float32)

    def blk(b, carry):
        mb = mask_ref[:, pl.ds(b * 512, 512)]
        ex = lax.dot_general(mb, tri, (((1,), (0,)), ((), ())),
                             preferred_element_type=jnp.float32)
        cnt_ref[:, pl.ds(b * 512, 512)] = ex + carry
        return carry + jnp.sum(mb, axis=1, keepdims=True)

    lax.fori_loop(0, T // 512, blk, jnp.zeros((E, 1), jnp.float32))

    cnt = cnt_ref[...]
    slot1 = jnp.sum(jnp.where(row == i1, cnt, 0.0), axis=0, keepdims=True)
    slot2 = jnp.sum(jnp.where(row == i2, cnt, 0.0), axis=0, keepdims=True)
    cap = float(C)
    v1 = slot1 < cap
    v2 = slot2 < cap
    d1 = jnp.where(v1, i1.astype(jnp.float32) * cap + slot1, float(EC))
    d2 = jnp.where(v2, i2.astype(jnp.float32) * cap + slot2, float(EC))
    meta_ref[0:1, :] = d1
    meta_ref[1:2, :] = d2
    meta_ref[2:3, :] = jnp.where(v1, g1, 0.0)
    meta_ref[3:4, :] = jnp.where(v2, g2, 0.0)
    meta_ref[4:8, :] = jnp.zeros((4, T), jnp.float32)


def _router(xf, epsT, wt, bt):
    return pl.pallas_call(
        _router_body,
        out_shape=jax.ShapeDtypeStruct((8, T), jnp.float32),
        scratch_shapes=[
            pltpu.VMEM((E, T), jnp.float32),
            pltpu.VMEM((E, T), jnp.float32),
        ],
    )(xf, epsT, wt, bt)


def _dispatch_body(destf_hbm, x_hbm, xbuf_hbm, dest_v, ord_v, rows_v, sem):
    wid = lax.axis_index("s") * NC + lax.axis_index("c")
    lo = wid * RPW
    pltpu.sync_copy(destf_hbm, dest_v)

    zero = jnp.zeros((16,), jnp.int32)

    def zb(i, _):
        ord_v[pl.ds(i * 16, 16)] = zero
        return 0

    lax.fori_loop(0, RPW // 16, zb, 0)

    iot = lax.broadcasted_iota(jnp.int32, (16,), 0)

    def body(i, _):
        d = dest_v[pl.ds(i * 16, 16)]
        j = iot + jnp.full((16,), i * 16, jnp.int32)
        t = j & (T - 1)
        dl = d - lo
        m = (dl >= 0) & (dl < RPW)
        dlc = jnp.where(m, dl, 0)
        plsc.store_scatter(ord_v, (dlc,), t, mask=m)
        return 0

    lax.fori_loop(0, 2 * T // 16, body, 0)

    for sub in range(RPW // GCH):
        cp = pltpu.async_copy(x_hbm.at[ord_v.at[pl.ds(sub * GCH, GCH)]],
                              rows_v, sem)
        cp.wait()
        pltpu.sync_copy(rows_v, xbuf_hbm.at[pl.ds(lo + sub * GCH, GCH)])


def _dispatch(dest_flat, xf):
    f = pl.kernel(
        _dispatch_body,
        out_type=jax.ShapeDtypeStruct((EC, D), jnp.float32),
        mesh=plsc.VectorSubcoreMesh(core_axis_name="c", subcore_axis_name="s"),
        compiler_params=pltpu.CompilerParams(needs_layout_passes=False),
        scratch_types=[
            pltpu.VMEM((2 * T,), jnp.int32),
            pltpu.VMEM((RPW,), jnp.int32),
            pltpu.VMEM((GCH, D), jnp.float32),
            pltpu.SemaphoreType.DMA,
        ],
    )
    return f(dest_flat, xf)


def _ffn_body(x_ref, w1_ref, b1_ref, w2_ref, b2_ref, y_ref, acc_ref):
    fb = pl.program_id(1)
    x = x_ref[...].astype(jnp.bfloat16)
    w1 = w1_ref[0].astype(jnp.bfloat16)
    z = lax.dot_general(x, w1, (((1,), (1,)), ((), ())),
                        preferred_element_type=jnp.float32)
    z = z + b1_ref[0]
    h = z * 0.5 * (1.0 + lax.erf(z * _INV_SQRT2))
    w2 = w2_ref[0].astype(jnp.bfloat16)
    p = lax.dot_general(h.astype(jnp.bfloat16), w2, (((1,), (1,)), ((), ())),
                        preferred_element_type=jnp.float32)

    @pl.when(fb == 0)
    def _():
        acc_ref[...] = p

    @pl.when(fb != 0)
    def _():
        acc_ref[...] = acc_ref[...] + p

    @pl.when(fb == NF - 1)
    def _():
        y_ref[...] = acc_ref[...] + b2_ref[0]


def _ffn(xbuf, w1, b1, w2, b2):
    return pl.pallas_call(
        _ffn_body,
        grid=(E, NF),
        in_specs=[
            pl.BlockSpec((C, D), lambda e, f: (e, 0)),
            pl.BlockSpec((1, FB, D), lambda e, f: (e, f, 0)),
            pl.BlockSpec((1, 1, FB), lambda e, f: (e * NF + f, 0, 0)),
            pl.BlockSpec((1, D, FB), lambda e, f: (e, 0, f)),
            pl.BlockSpec((1, 1, D), lambda e, f: (e, 0, 0)),
        ],
        out_specs=pl.BlockSpec((C, D), lambda e, f: (e, 0)),
        out_shape=jax.ShapeDtypeStruct((EC, D), jnp.float32),
        scratch_shapes=[pltpu.VMEM((C, D), jnp.float32)],
    )(xbuf, w1, b1.reshape(E * NF, 1, FB), w2, b2.reshape(E, 1, D))


def _combine_body(d0_hbm, d1_hbm, g0_hbm, g1_hbm, ybuf_hbm, out_hbm,
                  d0_v, d1_v, g0_v, g1_v, rows0, rows1, out_v, sem):
    wid = lax.axis_index("s") * NC + lax.axis_index("c")
    tbase = wid * TPW
    pltpu.sync_copy(d0_hbm.at[pl.ds(tbase, TPW)], d0_v)
    pltpu.sync_copy(d1_hbm.at[pl.ds(tbase, TPW)], d1_v)
    pltpu.sync_copy(g0_hbm.at[pl.ds(tbase, TPW)], g0_v.at[pl.ds(0, TPW)])
    pltpu.sync_copy(g1_hbm.at[pl.ds(tbase, TPW)], g1_v.at[pl.ds(0, TPW)])

    cap = jnp.full((16,), EC - 1, jnp.int32)

    def cl(i, _):
        sl = pl.ds(i * 16, 16)
        d0_v[sl] = jnp.minimum(d0_v[sl], cap)
        d1_v[sl] = jnp.minimum(d1_v[sl], cap)
        return 0

    lax.fori_loop(0, TPW // 16, cl, 0)

    for ch in range(TPW // CCH):
        cp0 = pltpu.async_copy(ybuf_hbm.at[d0_v.at[pl.ds(ch * CCH, CCH)]],
                               rows0, sem)
        cp1 = pltpu.async_copy(ybuf_hbm.at[d1_v.at[pl.ds(ch * CCH, CCH)]],
                               rows1, sem)
        cp0.wait()
        cp1.wait()

        def tok(tk, _):
            g0 = g0_v[pl.ds(ch * CCH + tk, 16)][0]
            g1 = g1_v[pl.ds(ch * CCH + tk, 16)][0]

            def vb(v, _):
                sl = pl.ds(v * 16, 16)
                out_v[tk, sl] = rows0[tk, sl] * g0 + rows1[tk, sl] * g1
                return 0

            lax.fori_loop(0, D // 16, vb, 0)
            return 0

        lax.fori_loop(0, CCH, tok, 0)
        pltpu.sync_copy(out_v, out_hbm.at[pl.ds(tbase + ch * CCH, CCH)])


def _combine(d0, d1, g0, g1, ybuf):
    f = pl.kernel(
        _combine_body,
        out_type=jax.ShapeDtypeStruct((T, D), jnp.float32),
        mesh=plsc.VectorSubcoreMesh(core_axis_name="c", subcore_axis_name="s"),
        scratch_types=[
            pltpu.VMEM((TPW,), jnp.int32),
            pltpu.VMEM((TPW,), jnp.int32),
            pltpu.VMEM((TPW + 16,), jnp.float32),
            pltpu.VMEM((TPW + 16,), jnp.float32),
            pltpu.VMEM((CCH, D), jnp.float32),
            pltpu.VMEM((CCH, D), jnp.float32),
            pltpu.VMEM((CCH, D), jnp.float32),
            pltpu.SemaphoreType.DMA,
        ],
    )
    return f(d0, d1, g0, g1, ybuf)


def kernel(x, noise_eps, router_w, router_b, noise_w, noise_b,
           fc1_w, fc1_b, fc2_w, fc2_b):
    xf = x.reshape(T, D)
    epsT = noise_eps.T
    wt = jnp.concatenate([router_w, noise_w], axis=0)
    bt = jnp.concatenate([router_b, noise_b])[:, None]

    meta = _router(xf, epsT, wt, bt)
    dest = meta[0:2].astype(jnp.int32)
    g = meta[2:4]

    xbuf = _dispatch(dest.reshape(-1), xf)
    ybuf = _ffn(xbuf, fc1_w, fc1_b, fc2_w, fc2_b)
    out = _combine(dest[0], dest[1], g[0], g[1], ybuf)
    return out.reshape(B, S, D)


# speedup vs baseline: 1.0799x; 1.0799x over previous
import functools
import math

import jax
import jax.numpy as jnp
from jax import lax
from jax.experimental import pallas as pl
from jax.experimental.pallas import tpu as pltpu
from jax.experimental.pallas import tpu_sc as plsc

B = 2
S = 2048
D = 1024
DFF = 4096
E = 16
TOPK = 2
T = B * S
C = 512
EC = E * C

NC = 2
NS = 16
NW = NC * NS
RPW = EC // NW
GCH = 32
TPW = T // NW
CCH = 16

FB = 2048
NF = DFF // FB

_INV_SQRT2 = 1.0 / math.sqrt(2.0)


def _router_body(xf_ref, epsT_ref, wt_ref, bt_ref, meta_ref, mask_ref, cnt_ref):
    xf = xf_ref[...]
    wt = wt_ref[...]
    lgT = lax.dot_general(wt, xf, (((1,), (1,)), ((), ())),
                          preferred_element_type=jnp.float32)
    lgT = lgT + bt_ref[...]
    logits = lgT[0:E, :]
    nlog = lgT[E:2 * E, :]
    sp = jnp.maximum(nlog, 0.0) + jnp.log1p(jnp.exp(-jnp.abs(nlog)))
    noisy = logits + epsT_ref[...] * sp

    row = lax.broadcasted_iota(jnp.int32, (E, T), 0)
    m1 = jnp.max(noisy, axis=0, keepdims=True)
    i1 = jnp.min(jnp.where(noisy == m1, row, E), axis=0, keepdims=True)
    noisy2 = jnp.where(row == i1, -jnp.inf, noisy)
    m2 = jnp.max(noisy2, axis=0, keepdims=True)
    i2 = jnp.min(jnp.where(noisy2 == m2, row, E), axis=0, keepdims=True)

    e21 = jnp.exp(m2 - m1)
    den = 1.0 + e21
    g1 = 1.0 / den
    g2 = e21 / den

    mask_ref[...] = ((row == i1) | (row == i2)).astype(jnp.float32)

    r5 = lax.broadcasted_iota(jnp.int32, (512, 512), 0)
    c5 = lax.broadcasted_iota(jnp.int32, (512, 512), 1)
    tri = (r5 < c5).astype(jnp.float32)

    def blk(b, carry):
        mb = mask_ref[:, pl.ds(b * 512, 512)]
        ex = lax.dot_general(mb, tri, (((1,), (0,)), ((), ())),
                             preferred_element_type=jnp.float32)
        cnt_ref[:, pl.ds(b * 512, 512)] = ex + carry
        return carry + jnp.sum(mb, axis=1, keepdims=True)

    lax.fori_loop(0, T // 512, blk, jnp.zeros((E, 1), jnp.float32))

    cnt = cnt_ref[...]
    slot1 = jnp.sum(jnp.where(row == i1, cnt, 0.0), axis=0, keepdims=True)
    slot2 = jnp.sum(jnp.where(row == i2, cnt, 0.0), axis=0, keepdims=True)
    cap = float(C)
    v1 = slot1 < cap
    v2 = slot2 < cap
    d1 = jnp.where(v1, i1.astype(jnp.float32) * cap + slot1, float(EC))
    d2 = jnp.where(v2, i2.astype(jnp.float32) * cap + slot2, float(EC))
    meta_ref[0:1, :] = d1
    meta_ref[1:2, :] = d2
    meta_ref[2:3, :] = jnp.where(v1, g1, 0.0)
    meta_ref[3:4, :] = jnp.where(v2, g2, 0.0)
    meta_ref[4:8, :] = jnp.zeros((4, T), jnp.float32)


def _router(xf, epsT, wt, bt):
    return pl.pallas_call(
        _router_body,
        out_shape=jax.ShapeDtypeStruct((8, T), jnp.float32),
        scratch_shapes=[
            pltpu.VMEM((E, T), jnp.float32),
            pltpu.VMEM((E, T), jnp.float32),
        ],
    )(xf, epsT, wt, bt)


def _dispatch_body(destf_hbm, x_hbm, xbuf_hbm, dest_v, ord_v, rows_v,
                   semA, semB, semO):
    wid = lax.axis_index("s") * NC + lax.axis_index("c")
    lo = wid * RPW
    pltpu.sync_copy(destf_hbm, dest_v)

    zero = jnp.zeros((16,), jnp.int32)

    def zb(i, _):
        ord_v[pl.ds(i * 16, 16)] = zero
        return 0

    lax.fori_loop(0, RPW // 16, zb, 0)

    iot = lax.broadcasted_iota(jnp.int32, (16,), 0)

    def body(i, _):
        d = dest_v[pl.ds(i * 16, 16)]
        j = iot + jnp.full((16,), i * 16, jnp.int32)
        t = j & (T - 1)
        dl = d - lo
        m = (dl >= 0) & (dl < RPW)
        dlc = jnp.where(m, dl, 0)
        plsc.store_scatter(ord_v, (dlc,), t, mask=m)
        return 0

    lax.fori_loop(0, 2 * T // 16, body, 0)

    nsub = RPW // GCH
    sems = (semA, semB)

    def fire(sub):
        p = sub & 1
        return pltpu.async_copy(x_hbm.at[ord_v.at[pl.ds(sub * GCH, GCH)]],
                                rows_v.at[p], sems[p])

    outc = [None, None]
    pend = fire(0)
    for sub in range(nsub):
        p = sub & 1
        nxt = None
        if sub + 1 < nsub:
            q = (sub + 1) & 1
            if outc[q] is not None:
                outc[q].wait()
                outc[q] = None
            nxt = fire(sub + 1)
        pend.wait()
        outc[p] = pltpu.async_copy(rows_v.at[p],
                                   xbuf_hbm.at[pl.ds(lo + sub * GCH, GCH)],
                                   semO)
        pend = nxt
    for c in outc:
        if c is not None:
            c.wait()


def _dispatch(dest_flat, xf):
    f = pl.kernel(
        _dispatch_body,
        out_type=jax.ShapeDtypeStruct((EC, D), jnp.float32),
        mesh=plsc.VectorSubcoreMesh(core_axis_name="c", subcore_axis_name="s"),
        compiler_params=pltpu.CompilerParams(needs_layout_passes=False),
        scratch_types=[
            pltpu.VMEM((2 * T,), jnp.int32),
            pltpu.VMEM((RPW,), jnp.int32),
            pltpu.VMEM((2, GCH, D), jnp.float32),
            pltpu.SemaphoreType.DMA,
            pltpu.SemaphoreType.DMA,
            pltpu.SemaphoreType.DMA,
        ],
    )
    return f(dest_flat, xf)


def _ffn_body(x_ref, w1_ref, b1_ref, w2_ref, b2_ref, y_ref, acc_ref):
    fb = pl.program_id(1)
    x = x_ref[...]
    w1 = w1_ref[0]
    z = lax.dot_general(x, w1, (((1,), (1,)), ((), ())),
                        preferred_element_type=jnp.float32)
    z = z + b1_ref[0]
    h = z * 0.5 * (1.0 + lax.erf(z * _INV_SQRT2))
    w2 = w2_ref[0]
    p = lax.dot_general(h, w2, (((1,), (1,)), ((), ())),
                        preferred_element_type=jnp.float32)

    @pl.when(fb == 0)
    def _():
        acc_ref[...] = p

    @pl.when(fb != 0)
    def _():
        acc_ref[...] = acc_ref[...] + p

    @pl.when(fb == NF - 1)
    def _():
        y_ref[...] = acc_ref[...] + b2_ref[0]


def _ffn(xbuf, w1, b1, w2, b2):
    return pl.pallas_call(
        _ffn_body,
        grid=(E, NF),
        in_specs=[
            pl.BlockSpec((C, D), lambda e, f: (e, 0)),
            pl.BlockSpec((1, FB, D), lambda e, f: (e, f, 0)),
            pl.BlockSpec((1, 1, FB), lambda e, f: (e * NF + f, 0, 0)),
            pl.BlockSpec((1, D, FB), lambda e, f: (e, 0, f)),
            pl.BlockSpec((1, 1, D), lambda e, f: (e, 0, 0)),
        ],
        out_specs=pl.BlockSpec((C, D), lambda e, f: (e, 0)),
        out_shape=jax.ShapeDtypeStruct((EC, D), jnp.float32),
        scratch_shapes=[pltpu.VMEM((C, D), jnp.float32)],
    )(xbuf, w1, b1.reshape(E * NF, 1, FB), w2, b2.reshape(E, 1, D))


def _combine_body(d0_hbm, d1_hbm, g0_hbm, g1_hbm, ybuf_hbm, out_hbm,
                  d0_v, d1_v, g0_v, g1_v, rows0, rows1, out_v,
                  semA, semB, semO):
    wid = lax.axis_index("s") * NC + lax.axis_index("c")
    tbase = wid * TPW
    pltpu.sync_copy(d0_hbm.at[pl.ds(tbase, TPW)], d0_v)
    pltpu.sync_copy(d1_hbm.at[pl.ds(tbase, TPW)], d1_v)
    pltpu.sync_copy(g0_hbm.at[pl.ds(tbase, TPW)], g0_v.at[pl.ds(0, TPW)])
    pltpu.sync_copy(g1_hbm.at[pl.ds(tbase, TPW)], g1_v.at[pl.ds(0, TPW)])

    cap = jnp.full((16,), EC - 1, jnp.int32)

    def cl(i, _):
        sl = pl.ds(i * 16, 16)
        d0_v[sl] = jnp.minimum(d0_v[sl], cap)
        d1_v[sl] = jnp.minimum(d1_v[sl], cap)
        return 0

    lax.fori_loop(0, TPW // 16, cl, 0)

    nch = TPW // CCH
    sems = (semA, semB)

    def fire(ch):
        p = ch & 1
        c0 = pltpu.async_copy(ybuf_hbm.at[d0_v.at[pl.ds(ch * CCH, CCH)]],
                              rows0.at[p], sems[p])
        c1 = pltpu.async_copy(ybuf_hbm.at[d1_v.at[pl.ds(ch * CCH, CCH)]],
                              rows1.at[p], sems[p])
        return (c0, c1)

    outc = [None, None]
    pend = fire(0)
    for ch in range(nch):
        p = ch & 1
        nxt = None
        if ch + 1 < nch:
            nxt = fire(ch + 1)
        pend[0].wait()
        pend[1].wait()
        if outc[p] is not None:
            outc[p].wait()
            outc[p] = None

        def tok(tk, _):
            g0 = g0_v[pl.ds(ch * CCH + tk, 16)][0]
            g1 = g1_v[pl.ds(ch * CCH + tk, 16)][0]

            def vb(v, _):
                sl = pl.ds(v * 16, 16)
                out_v[p, tk, sl] = rows0[p, tk, sl] * g0 + rows1[p, tk, sl] * g1
                return 0

            lax.fori_loop(0, D // 16, vb, 0)
            return 0

        lax.fori_loop(0, CCH, tok, 0)
        outc[p] = pltpu.async_copy(out_v.at[p],
                                   out_hbm.at[pl.ds(tbase + ch * CCH, CCH)],
                                   semO)
        pend = nxt
    for c in outc:
        if c is not None:
            c.wait()


def _combine(d0, d1, g0, g1, ybuf):
    f = pl.kernel(
        _combine_body,
        out_type=jax.ShapeDtypeStruct((T, D), jnp.float32),
        mesh=plsc.VectorSubcoreMesh(core_axis_name="c", subcore_axis_name="s"),
        scratch_types=[
            pltpu.VMEM((TPW,), jnp.int32),
            pltpu.VMEM((TPW,), jnp.int32),
            pltpu.VMEM((TPW + 16,), jnp.float32),
            pltpu.VMEM((TPW + 16,), jnp.float32),
            pltpu.VMEM((2, CCH, D), jnp.float32),
            pltpu.VMEM((2, CCH, D), jnp.float32),
            pltpu.VMEM((2, CCH, D), jnp.float32),
            pltpu.SemaphoreType.DMA,
            pltpu.SemaphoreType.DMA,
            pltpu.SemaphoreType.DMA,
        ],
    )
    return f(d0, d1, g0, g1, ybuf)


def kernel(x, noise_eps, router_w, router_b, noise_w, noise_b,
           fc1_w, fc1_b, fc2_w, fc2_b):
    xf = x.reshape(T, D)
    epsT = noise_eps.T
    wt = jnp.concatenate([router_w, noise_w], axis=0)
    bt = jnp.concatenate([router_b, noise_b])[:, None]

    meta = _router(xf, epsT, wt, bt)
    dest = meta[0:2].astype(jnp.int32)
    g = meta[2:4]

    xbuf = _dispatch(dest.reshape(-1), xf)
    ybuf = _ffn(xbuf, fc1_w, fc1_b, fc2_w, fc2_b)
    out = _combine(dest[0], dest[1], g[0], g[1], ybuf)
    return out.reshape(B, S, D)
